# Initial kernel scaffold; baseline (speedup 1.0000x reference)
import jax
import jax.numpy as jnp
from jax.experimental import pallas as pl


def kernel(x, w1, b1, w2, b2):
    raise NotImplementedError("write your pallas kernel here")


# speedup vs baseline: 1.3258x; 1.3258x over previous
import functools

import jax
import jax.numpy as jnp
from jax.experimental import pallas as pl
from jax.experimental.pallas import tpu as pltpu

_PACK = 16
_IN_F = 8
_HID = 12


def _mlp_tile(x_ref, w1_ref, b1_ref, w2_ref, b2_ref, o_ref):
    h = jax.lax.dot_general(
        x_ref[...], w1_ref[...],
        dimension_numbers=(((1,), (0,)), ((), ())),
        preferred_element_type=jnp.float32,
        precision=jax.lax.Precision.DEFAULT,
    )
    h = jnp.maximum(h + b1_ref[...], 0.0)
    z = jax.lax.dot_general(
        h, w2_ref[...],
        dimension_numbers=(((1,), (0,)), ((), ())),
        preferred_element_type=jnp.float32,
        precision=jax.lax.Precision.DEFAULT,
    )
    o_ref[...] = jax.nn.sigmoid(z + b2_ref[0])


@functools.partial(jax.jit, static_argnames=("tile_rows",))
def _forward(x, w1, b1, w2, b2, *, tile_rows=2048):
    B = x.shape[0]
    rows = B // _PACK
    rem = B % _PACK
    if rem:
        x = jnp.concatenate(
            [x, jnp.zeros((_PACK - rem, _IN_F), x.dtype)], axis=0)
        rows += 1
    x_r = x.astype(jnp.float32).reshape(rows, _PACK * _IN_F)

    eye = jnp.eye(_PACK, dtype=jnp.float32)
    w1big = jnp.kron(eye, w1.astype(jnp.float32).T)
    b1big = jnp.tile(b1.astype(jnp.float32), _PACK)[None, :]
    w2big = jnp.kron(eye, w2.astype(jnp.float32).T)
    b2s = b2.astype(jnp.float32).reshape(1)

    tbr = min(tile_rows, rows)
    grid = (pl.cdiv(rows, tbr),)

    out_r = pl.pallas_call(
        _mlp_tile,
        out_shape=jax.ShapeDtypeStruct((rows, _PACK), jnp.float32),
        grid=grid,
        in_specs=[
            pl.BlockSpec((tbr, _PACK * _IN_F), lambda i: (i, 0)),
            pl.BlockSpec((_PACK * _IN_F, _PACK * _HID), lambda i: (0, 0)),
            pl.BlockSpec((1, _PACK * _HID), lambda i: (0, 0)),
            pl.BlockSpec((_PACK * _HID, _PACK), lambda i: (0, 0)),
            pl.BlockSpec(memory_space=pltpu.MemorySpace.SMEM),
        ],
        out_specs=pl.BlockSpec((tbr, _PACK), lambda i: (i, 0)),
        compiler_params=pltpu.CompilerParams(
            dimension_semantics=("parallel",),
            vmem_limit_bytes=64 * 1024 * 1024,
        ),
    )(x_r, w1big, b1big, w2big, b2s)

    out = out_r.reshape(rows * _PACK, 1)
    return out[:B] if rem else out


def kernel(x, w1, b1, w2, b2):
    return _forward(x, w1, b1, w2, b2)


# speedup vs baseline: 17.0113x; 12.8306x over previous
import functools

import jax
import jax.numpy as jnp
from jax.experimental import pallas as pl
from jax.experimental.pallas import tpu as pltpu

_IN_F = 8
_HID = 12


def _mlp_lanes(xt_ref, w1_ref, b1_ref, w2_ref, b2_ref, o_ref):
    feats = [xt_ref[k, :] for k in range(_IN_F)]
    z = jnp.full_like(feats[0], b2_ref[0])
    for j in range(_HID):
        h = feats[0] * w1_ref[j, 0]
        for k in range(1, _IN_F):
            h = h + feats[k] * w1_ref[j, k]
        h = jnp.maximum(h + b1_ref[j], 0.0)
        z = z + h * w2_ref[j]
    o_ref[...] = jax.nn.sigmoid(z)


@functools.partial(jax.jit, static_argnames=("lane_block",))
def _forward(x, w1, b1, w2, b2, *, lane_block=65536):
    B = x.shape[0]
    xt = x.astype(jnp.float32).T
    pad = -B % 128
    if pad:
        xt = jnp.pad(xt, ((0, 0), (0, pad)))
    n = xt.shape[1]

    lb = min(lane_block, n)
    grid = (pl.cdiv(n, lb),)

    out_flat = pl.pallas_call(
        _mlp_lanes,
        out_shape=jax.ShapeDtypeStruct((n,), jnp.float32),
        grid=grid,
        in_specs=[
            pl.BlockSpec((_IN_F, lb), lambda i: (0, i)),
            pl.BlockSpec(memory_space=pltpu.MemorySpace.SMEM),
            pl.BlockSpec(memory_space=pltpu.MemorySpace.SMEM),
            pl.BlockSpec(memory_space=pltpu.MemorySpace.SMEM),
            pl.BlockSpec(memory_space=pltpu.MemorySpace.SMEM),
        ],
        out_specs=pl.BlockSpec((lb,), lambda i: (i,)),
        compiler_params=pltpu.CompilerParams(
            dimension_semantics=("parallel",),
        ),
    )(
        xt,
        w1.astype(jnp.float32),
        b1.astype(jnp.float32),
        w2.astype(jnp.float32).reshape(_HID),
        b2.astype(jnp.float32),
    )

    return out_flat[:B].reshape(B, 1)


def kernel(x, w1, b1, w2, b2):
    return _forward(x, w1, b1, w2, b2)


# speedup vs baseline: 20.0926x; 1.1811x over previous
import functools

import jax
import jax.numpy as jnp
from jax.experimental import pallas as pl
from jax.experimental.pallas import tpu as pltpu

_IN_F = 8
_HID = 12


def _mlp_mxu2(xt_ref, w1_ref, b1_ref, w2_ref, b2_ref, o_ref):
    x = xt_ref[...]
    h = jax.lax.dot_general(
        w1_ref[...], x,
        dimension_numbers=(((1,), (0,)), ((), ())),
        preferred_element_type=jnp.float32,
        precision=jax.lax.Precision.DEFAULT,
    )
    h = jnp.maximum(h + b1_ref[...], 0.0)
    z8 = jax.lax.dot_general(
        w2_ref[...], h,
        dimension_numbers=(((1,), (0,)), ((), ())),
        preferred_element_type=jnp.float32,
        precision=jax.lax.Precision.DEFAULT,
    )
    o_ref[...] = jax.nn.sigmoid(z8[0, :] + b2_ref[0])


@functools.partial(jax.jit, static_argnames=("lane_block",))
def _forward(x, w1, b1, w2, b2, *, lane_block=65536):
    B = x.shape[0]
    xt = x.astype(jnp.float32).T
    pad = -B % 128
    if pad:
        xt = jnp.pad(xt, ((0, 0), (0, pad)))
    n = xt.shape[1]

    w1p = jnp.zeros((16, _IN_F), jnp.float32).at[:_HID].set(
        w1.astype(jnp.float32))
    b1p = jnp.zeros((16, 1), jnp.float32).at[:_HID, 0].set(
        b1.astype(jnp.float32))
    w2p = jnp.tile(
        jnp.zeros((1, 16), jnp.float32).at[0, :_HID].set(
            w2.astype(jnp.float32).reshape(_HID)),
        (8, 1))

    lb = min(lane_block, n)
    grid = (pl.cdiv(n, lb),)

    out_flat = pl.pallas_call(
        _mlp_mxu2,
        out_shape=jax.ShapeDtypeStruct((n,), jnp.float32),
        grid=grid,
        in_specs=[
            pl.BlockSpec((_IN_F, lb), lambda i: (0, i)),
            pl.BlockSpec((16, _IN_F), lambda i: (0, 0)),
            pl.BlockSpec((16, 1), lambda i: (0, 0)),
            pl.BlockSpec((8, 16), lambda i: (0, 0)),
            pl.BlockSpec(memory_space=pltpu.MemorySpace.SMEM),
        ],
        out_specs=pl.BlockSpec((lb,), lambda i: (i,)),
        compiler_params=pltpu.CompilerParams(
            dimension_semantics=("parallel",),
        ),
    )(
        xt,
        w1p,
        b1p,
        w2p,
        b2.astype(jnp.float32),
    )

    return out_flat[:B].reshape(B, 1)


def kernel(x, w1, b1, w2, b2):
    return _forward(x, w1, b1, w2, b2)


# speedup vs baseline: 22.5954x; 1.1246x over previous
import functools

import jax
import jax.numpy as jnp
from jax.experimental import pallas as pl
from jax.experimental.pallas import tpu as pltpu

_IN_F = 8
_HID = 12


def _mlp_mxu2(xt_ref, w1_ref, b1_ref, w2_ref, b2_ref, o_ref):
    x = xt_ref[...]
    h = jax.lax.dot_general(
        w1_ref[...], x,
        dimension_numbers=(((1,), (0,)), ((), ())),
        preferred_element_type=jnp.float32,
        precision=jax.lax.Precision.DEFAULT,
    )
    h = jnp.maximum(h + b1_ref[...], 0.0)
    z8 = jax.lax.dot_general(
        w2_ref[...], h,
        dimension_numbers=(((1,), (0,)), ((), ())),
        preferred_element_type=jnp.float32,
        precision=jax.lax.Precision.DEFAULT,
    )
    o_ref[...] = jax.nn.sigmoid(z8[0, :] + b2_ref[0])


@functools.partial(jax.jit, static_argnames=("lane_block",))
def _forward(x, w1, b1, w2, b2, *, lane_block=131072):
    B = x.shape[0]
    xt = x.astype(jnp.float32).T
    pad = -B % 128
    if pad:
        xt = jnp.pad(xt, ((0, 0), (0, pad)))
    n = xt.shape[1]

    w1p = jnp.zeros((16, _IN_F), jnp.float32).at[:_HID].set(
        w1.astype(jnp.float32))
    b1p = jnp.zeros((16, 1), jnp.float32).at[:_HID, 0].set(
        b1.astype(jnp.float32))
    w2p = jnp.tile(
        jnp.zeros((1, 16), jnp.float32).at[0, :_HID].set(
            w2.astype(jnp.float32).reshape(_HID)),
        (8, 1))

    lb = min(lane_block, n)
    grid = (pl.cdiv(n, lb),)

    out_flat = pl.pallas_call(
        _mlp_mxu2,
        out_shape=jax.ShapeDtypeStruct((n,), jnp.float32),
        grid=grid,
        in_specs=[
            pl.BlockSpec((_IN_F, lb), lambda i: (0, i)),
            pl.BlockSpec((16, _IN_F), lambda i: (0, 0)),
            pl.BlockSpec((16, 1), lambda i: (0, 0)),
            pl.BlockSpec((8, 16), lambda i: (0, 0)),
            pl.BlockSpec(memory_space=pltpu.MemorySpace.SMEM),
        ],
        out_specs=pl.BlockSpec((lb,), lambda i: (i,)),
        compiler_params=pltpu.CompilerParams(
            dimension_semantics=("parallel",),
        ),
    )(
        xt,
        w1p,
        b1p,
        w2p,
        b2.astype(jnp.float32),
    )

    return out_flat[:B].reshape(B, 1)


def kernel(x, w1, b1, w2, b2):
    return _forward(x, w1, b1, w2, b2)
